# Initial kernel scaffold; baseline (speedup 1.0000x reference)
import jax
import jax.numpy as jnp
from jax.experimental import pallas as pl


def kernel(inputs, pooling_indices, output_shape):
    raise NotImplementedError("write your pallas kernel here")


# speedup vs baseline: 12.6714x; 12.6714x over previous
import jax
import jax.numpy as jnp
from jax import lax
from jax.experimental import pallas as pl
from jax.experimental.pallas import tpu as pltpu
from jax.experimental.pallas import tpu_sc as plsc

B, H, W, C = 8, 112, 112, 96
HO, WO = 224, 224
CW = 16
NCG = C // CW
NSUB = 16
HPT = H // NSUB
YPT = HO // NSUB
POS = HPT * W
NROW = POS // 8
CHUNK = HO * WO * CW
SLICE = CHUNK // NSUB
NZ = SLICE // 4
ROWW = WO * CW


def _body(x_hbm, idx_hbm, zo_hbm, out_hbm,
          raw_val, raw_idx, svals, soffs, zeros, zo_v, chunk, sem_in, sem_sc):
    c = lax.axis_index("c")
    s = lax.axis_index("s")
    lane = lax.iota(jnp.int32, 16)

    @pl.loop(0, NZ // 16)
    def _zero_init(i):
        zeros[pl.ds(i * 16, 16)] = jnp.zeros((16,), jnp.float32)

    pltpu.sync_copy(zo_hbm, zo_v)
    zo = zo_v[...]

    for z in range(4):
        pltpu.sync_copy(zeros, chunk.at[pl.ds(s * SLICE + z * NZ, NZ)])

    @pl.loop(0, B // 2 * NCG)
    def _chunk_loop(k):
        b = c * (B // 2) + k // NCG
        cg = k - (k // NCG) * NCG
        c0 = cg * CW
        h0 = s * HPT

        cp_v = pltpu.async_copy(
            x_hbm.at[b, pl.ds(h0, HPT), :, pl.ds(c0, CW)], raw_val, sem_in)
        cp_i = pltpu.async_copy(
            idx_hbm.at[b, pl.ds(h0, HPT), :, pl.ds(c0, CW)], raw_idx, sem_in)
        cp_v.wait()
        cp_i.wait()

        @pl.loop(0, HPT)
        def _h_loop(hh):
            @pl.loop(0, W // 8)
            def _w_loop(wb):
                row = hh * (W // 8) + wb
                for p in range(8):
                    vi = raw_idx[hh, wb * 8 + p, :]
                    vv = raw_val[hh, wb * 8 + p, :]
                    q = lax.div(lax.add(vi, zo), jnp.full((16,), C, jnp.int32))
                    off = lax.add(lax.mul(q, jnp.full((16,), CW, jnp.int32)),
                                  lane)
                    soffs[row, pl.ds(p * 16, 16)] = off
                    svals[row, pl.ds(p * 16, 16)] = vv

        plsc.subcore_barrier()

        @pl.loop(0, NROW)
        def _fire(r):
            pltpu.async_copy(svals.at[r], chunk.at[soffs.at[r]], sem_sc,
                             add=True)

        @pl.loop(0, NROW)
        def _drain(r):
            pltpu.make_async_copy(svals.at[0], chunk.at[soffs.at[0]],
                                  sem_sc).wait()

        plsc.subcore_barrier()

        y0 = s * YPT

        @pl.loop(0, YPT)
        def _flush(yy):
            pltpu.sync_copy(
                chunk.at[pl.ds((y0 + yy) * ROWW, ROWW)],
                out_hbm.at[cg, b, y0 + yy, :])

        for z in range(4):
            pltpu.sync_copy(zeros, chunk.at[pl.ds(s * SLICE + z * NZ, NZ)])


@jax.jit
def _unpool(x, idx, zo16):
    fn = pl.kernel(
        _body,
        out_type=jax.ShapeDtypeStruct((NCG, B, HO, ROWW), jnp.float32),
        mesh=plsc.VectorSubcoreMesh(core_axis_name="c", subcore_axis_name="s"),
        compiler_params=pltpu.CompilerParams(use_tc_tiling_on_sc=False),
        scratch_types=[
            pltpu.VMEM((HPT, W, CW), jnp.float32),
            pltpu.VMEM((HPT, W, CW), jnp.int32),
            pltpu.VMEM((NROW, 128), jnp.float32),
            pltpu.VMEM((NROW, 128), jnp.int32),
            pltpu.VMEM((NZ,), jnp.float32),
            pltpu.VMEM((16,), jnp.int32),
            pltpu.VMEM_SHARED((CHUNK,), jnp.float32),
            pltpu.SemaphoreType.DMA,
            pltpu.SemaphoreType.DMA,
        ],
    )
    out_t = fn(x, idx, zo16)
    return (out_t.reshape(NCG, B, HO, WO, CW)
            .transpose(1, 2, 3, 0, 4)
            .reshape(B, HO, WO, C))


def kernel(inputs, pooling_indices, output_shape):
    shape_arr = jnp.asarray(output_shape).astype(jnp.int32)
    zo = jnp.sum(shape_arr) - jnp.int32(B + HO + WO + C)
    zo16 = jnp.broadcast_to(zo, (16,)).astype(jnp.int32)
    return _unpool(inputs, pooling_indices.astype(jnp.int32), zo16)


# speedup vs baseline: 13.7126x; 1.0822x over previous
import jax
import jax.numpy as jnp
from jax import lax
from jax.experimental import pallas as pl
from jax.experimental.pallas import tpu as pltpu
from jax.experimental.pallas import tpu_sc as plsc

B, H, W, C = 8, 112, 112, 96
HO, WO = 224, 224
CW = 16
NCG = C // CW
NSUB = 16
HPT = H // NSUB
YPT = HO // NSUB
POS = HPT * W
NROW = POS // 8
CHUNK = HO * WO * CW
SLICE = CHUNK // NSUB
NZ = SLICE // 16
ROWW = WO * CW


NCHUNK = B // 2 * NCG


def _body(x_hbm, idx_hbm, zo_hbm, out_hbm,
          raw_val, raw_idx, svals, soffs, zeros, zo_v, chunk,
          sem_in, sem_sc, sem_fl):
    c = lax.axis_index("c")
    s = lax.axis_index("s")
    lane = lax.iota(jnp.int32, 16)
    h0 = s * HPT
    y0 = s * YPT

    @pl.loop(0, NZ // 16)
    def _zero_init(i):
        zeros[pl.ds(i * 16, 16)] = jnp.zeros((16,), jnp.float32)

    pltpu.sync_copy(zo_hbm, zo_v)
    zo = zo_v[...]

    def fire_loads(k, p):
        b = c * (B // 2) + k // NCG
        cg = k - (k // NCG) * NCG
        pltpu.async_copy(
            x_hbm.at[cg, b, pl.ds(h0, HPT), :], raw_val.at[p], sem_in)
        pltpu.async_copy(
            idx_hbm.at[cg, b, pl.ds(h0, HPT), :], raw_idx.at[p], sem_in)

    def wait_loads(p):
        pltpu.make_async_copy(
            x_hbm.at[0, 0, pl.ds(0, HPT), :], raw_val.at[p], sem_in).wait()
        pltpu.make_async_copy(
            idx_hbm.at[0, 0, pl.ds(0, HPT), :], raw_idx.at[p], sem_in).wait()

    fire_loads(0, 0)
    for z in range(16):
        pltpu.sync_copy(zeros, chunk.at[pl.ds(s * SLICE + z * NZ, NZ)])
    plsc.subcore_barrier()

    @pl.loop(0, NCHUNK)
    def _chunk_loop(k):
        b = c * (B // 2) + k // NCG
        cg = k - (k // NCG) * NCG
        p = k - (k // 2) * 2

        wait_loads(p)

        @pl.when(k < NCHUNK - 1)
        def _prefetch():
            fire_loads(k + 1, 1 - p)

        @pl.loop(0, HPT)
        def _h_loop(hh):
            @pl.loop(0, W // 8)
            def _w_loop(wb):
                row = hh * (W // 8) + wb
                for pp in range(8):
                    vi = raw_idx[p, hh, pl.ds((wb * 8 + pp) * CW, 16)]
                    vv = raw_val[p, hh, pl.ds((wb * 8 + pp) * CW, 16)]
                    q = lax.div(lax.add(vi, zo), jnp.full((16,), C, jnp.int32))
                    off = lax.add(lax.mul(q, jnp.full((16,), CW, jnp.int32)),
                                  lane)
                    soffs[row, pl.ds(pp * 16, 16)] = off
                    svals[row, pl.ds(pp * 16, 16)] = vv
                pltpu.async_copy(svals.at[row], chunk.at[soffs.at[row]],
                                 sem_sc, add=True)

        @pl.loop(0, NROW)
        def _drain(r):
            pltpu.make_async_copy(svals.at[0], chunk.at[soffs.at[0]],
                                  sem_sc).wait()

        plsc.subcore_barrier()

        @pl.loop(0, YPT)
        def _flush_fire(yy):
            pltpu.async_copy(chunk.at[pl.ds((y0 + yy) * ROWW, ROWW)],
                             out_hbm.at[cg, b, y0 + yy, :], sem_fl)

        @pl.loop(0, YPT)
        def _flush_drain(yy):
            pltpu.make_async_copy(chunk.at[pl.ds(y0 * ROWW, ROWW)],
                                  out_hbm.at[cg, b, y0, :], sem_fl).wait()

        @pl.loop(0, 16)
        def _zero_fire(z):
            pltpu.async_copy(zeros, chunk.at[pl.ds(s * SLICE + z * NZ, NZ)],
                             sem_fl)

        @pl.loop(0, 16)
        def _zero_drain(z):
            pltpu.make_async_copy(zeros, chunk.at[pl.ds(s * SLICE, NZ)],
                                  sem_fl).wait()

        plsc.subcore_barrier()


@jax.jit
def _unpool(x, idx, zo16):
    x = (x.reshape(B, H, W, NCG, CW).transpose(3, 0, 1, 2, 4)
         .reshape(NCG, B, H, W * CW))
    idx = (idx.reshape(B, H, W, NCG, CW).transpose(3, 0, 1, 2, 4)
           .reshape(NCG, B, H, W * CW))
    fn = pl.kernel(
        _body,
        out_type=jax.ShapeDtypeStruct((NCG, B, HO, ROWW), jnp.float32),
        mesh=plsc.VectorSubcoreMesh(core_axis_name="c", subcore_axis_name="s"),
        compiler_params=pltpu.CompilerParams(use_tc_tiling_on_sc=False),
        scratch_types=[
            pltpu.VMEM((2, HPT, W * CW), jnp.float32),
            pltpu.VMEM((2, HPT, W * CW), jnp.int32),
            pltpu.VMEM((NROW, 128), jnp.float32),
            pltpu.VMEM((NROW, 128), jnp.int32),
            pltpu.VMEM((NZ,), jnp.float32),
            pltpu.VMEM((16,), jnp.int32),
            pltpu.VMEM_SHARED((CHUNK,), jnp.float32),
            pltpu.SemaphoreType.DMA,
            pltpu.SemaphoreType.DMA,
            pltpu.SemaphoreType.DMA,
        ],
    )
    out_t = fn(x, idx, zo16)
    return (out_t.reshape(NCG, B, HO, WO, CW)
            .transpose(1, 2, 3, 0, 4)
            .reshape(B, HO, WO, C))


def kernel(inputs, pooling_indices, output_shape):
    shape_arr = jnp.asarray(output_shape).astype(jnp.int32)
    zo = jnp.sum(shape_arr) - jnp.int32(B + HO + WO + C)
    zo16 = jnp.broadcast_to(zo, (16,)).astype(jnp.int32)
    return _unpool(inputs, pooling_indices.astype(jnp.int32), zo16)
